# Initial kernel scaffold; baseline (speedup 1.0000x reference)
import jax
import jax.numpy as jnp
from jax.experimental import pallas as pl


def kernel(features, A, B, b_final):
    raise NotImplementedError("write your pallas kernel here")


# speedup vs baseline: 1.0098x; 1.0098x over previous
import functools

import jax
import jax.numpy as jnp
from jax.experimental import pallas as pl

B_SZ, I, F, H = 8192, 16, 128, 32
BB = 512


def _fused_body(feat_ref, A_ref, B_ref, bf_ref,
                out_ref, h0_ref, h1_ref, hid_ref, pre_ref, nA_ref):
    dot = functools.partial(jax.lax.dot_general,
                            preferred_element_type=jnp.float32)
    for i in range(I):
        x = feat_ref[:, i * F:(i + 1) * F]
        A_i = A_ref[i]
        B_i = B_ref[i]

        inv = jax.lax.rsqrt(jnp.sum(A_i * A_i, axis=0, keepdims=True))
        nA = A_i * inv
        nA_ref[i] = nA

        h0 = dot(x, nA, (((1,), (0,)), ((), ())))
        hid = dot(h0, B_i, (((1,), (0,)), ((), ())))
        h1 = dot(hid, B_i, (((1,), (1,)), ((), ())))
        h2 = dot(h1, nA, (((1,), (1,)), ((), ())))
        pre = h2 + bf_ref[:, i * F:(i + 1) * F]

        h0_ref[:, i * F:(i + 1) * F] = h0
        hid_ref[:, i * H:(i + 1) * H] = hid
        h1_ref[:, i * F:(i + 1) * F] = h1
        pre_ref[:, i * F:(i + 1) * F] = pre
        out_ref[:, i * F:(i + 1) * F] = jnp.maximum(pre, 0.0)


def kernel(features, A, B, b_final):
    nbb = B_SZ // BB
    grid = (nbb,)

    batch_spec = pl.BlockSpec((BB, I * F), lambda b: (b, 0))
    const3 = pl.BlockSpec((I, F, F), lambda b: (0, 0, 0))

    in_specs = (
        batch_spec,
        const3,
        pl.BlockSpec((I, F, H), lambda b: (0, 0, 0)),
        pl.BlockSpec((1, I * F), lambda b: (0, 0)),
    )
    out_specs = (
        batch_spec,
        batch_spec,
        batch_spec,
        pl.BlockSpec((BB, I * H), lambda b: (b, 0)),
        batch_spec,
        const3,
    )
    out_shape = (
        jax.ShapeDtypeStruct((B_SZ, I * F), jnp.float32),
        jax.ShapeDtypeStruct((B_SZ, I * F), jnp.float32),
        jax.ShapeDtypeStruct((B_SZ, I * F), jnp.float32),
        jax.ShapeDtypeStruct((B_SZ, I * H), jnp.float32),
        jax.ShapeDtypeStruct((B_SZ, I * F), jnp.float32),
        jax.ShapeDtypeStruct((I, F, F), jnp.float32),
    )

    out, h0, h1, hid, pre, nA = pl.pallas_call(
        _fused_body,
        grid=grid,
        in_specs=in_specs,
        out_specs=out_specs,
        out_shape=out_shape,
    )(features.reshape(B_SZ, I * F), A, B, b_final.reshape(1, I * F))

    shp = (B_SZ, I, F)
    return (out.reshape(shp), h0.reshape(shp), h1.reshape(shp),
            hid.reshape(B_SZ, I, H), pre.reshape(shp), nA)


# speedup vs baseline: 1.2871x; 1.2747x over previous
import functools

import jax
import jax.numpy as jnp
from jax.experimental import pallas as pl

B_SZ, I, F, H = 8192, 16, 128, 32
BB = 512


def _fused_body(feat_ref, A_ref, B_ref, bf_ref,
                out_ref, h0_ref, h1_ref, hid_ref, pre_ref, nA_ref):
    dot = functools.partial(jax.lax.dot_general,
                            preferred_element_type=jnp.float32)
    for i in range(I):
        x = feat_ref[:, i, :]
        A_i = A_ref[i]
        B_i = B_ref[i]

        inv = jax.lax.rsqrt(jnp.sum(A_i * A_i, axis=0, keepdims=True))
        nA = A_i * inv
        nA_ref[i] = nA

        h0 = dot(x, nA, (((1,), (0,)), ((), ())))
        hid = dot(h0, B_i, (((1,), (0,)), ((), ())))
        h1 = dot(hid, B_i, (((1,), (1,)), ((), ())))
        h2 = dot(h1, nA, (((1,), (1,)), ((), ())))
        pre = h2 + bf_ref[i][None, :]

        h0_ref[:, i, :] = h0
        hid_ref[:, i, :] = hid
        h1_ref[:, i, :] = h1
        pre_ref[:, i, :] = pre
        out_ref[:, i, :] = jnp.maximum(pre, 0.0)


def kernel(features, A, B, b_final):
    nbb = B_SZ // BB
    grid = (nbb,)

    batch_spec = pl.BlockSpec((BB, I, F), lambda b: (b, 0, 0))
    const3 = pl.BlockSpec((I, F, F), lambda b: (0, 0, 0))

    in_specs = (
        batch_spec,
        const3,
        pl.BlockSpec((I, F, H), lambda b: (0, 0, 0)),
        pl.BlockSpec((I, F), lambda b: (0, 0)),
    )
    out_specs = (
        batch_spec,
        batch_spec,
        batch_spec,
        pl.BlockSpec((BB, I, H), lambda b: (b, 0, 0)),
        batch_spec,
        const3,
    )
    out_shape = (
        jax.ShapeDtypeStruct((B_SZ, I, F), jnp.float32),
        jax.ShapeDtypeStruct((B_SZ, I, F), jnp.float32),
        jax.ShapeDtypeStruct((B_SZ, I, F), jnp.float32),
        jax.ShapeDtypeStruct((B_SZ, I, H), jnp.float32),
        jax.ShapeDtypeStruct((B_SZ, I, F), jnp.float32),
        jax.ShapeDtypeStruct((I, F, F), jnp.float32),
    )

    return pl.pallas_call(
        _fused_body,
        grid=grid,
        in_specs=in_specs,
        out_specs=out_specs,
        out_shape=out_shape,
    )(features, A, B, b_final)
